# baseline (speedup 1.0000x reference)
import functools

import jax
import jax.numpy as jnp
from jax import lax
from jax.experimental import pallas as pl
from jax.experimental.pallas import tpu as pltpu
from jax.experimental.pallas import tpu_sc as plsc

H = 64
HH = 32
NATOM = 50000
NPAD = 50176
NBOND = 400000
NBPAD = 400384
NMOTIF = 40000
NMPAD = 40448
EBPAD = 803328
EMPAD = 400896
IDXPAD = 96
NGRAPH = 1024
GPAD = 1152
NSUB = 16
ROWS_T = NPAD // NSUB
GROWS_T = GPAD // NSUB
CHUNK = 96

_LOG1P = (
    3.5075520539162852e-06, 0.9997924357286234, -0.4969779111677995,
    0.31459053537163134, -0.18878267362227552, 0.0817268083763448,
    -0.017208061121549833,
)
_NDEG = len(_LOG1P) - 1

@functools.cache
def _mesh():
    return plsc.VectorSubcoreMesh(core_axis_name="c", subcore_axis_name="s",
                                  num_cores=2, num_subcores=NSUB)


def _gate(a, b):
    sig = 1.0 / (1.0 + jnp.exp(-a))
    t = jnp.exp(-jnp.abs(b))
    p = jnp.full_like(t, _LOG1P[_NDEG])
    for k in range(_NDEG - 1, -1, -1):
        p = p * t + _LOG1P[k]
    sp = jnp.maximum(b, 0.0) + p
    return sig * sp


def _sc_conv_body(nchunks, xt, ht, ni, hi, zeros32, out,
                  idx_n, idx_h, idxg_n, idxg_h, sidx, bufa, bufb, msg,
                  acc, sem_i0, sem_i1, sem_a0, sem_a1, sem_b0, sem_b1):
    c = lax.axis_index("c")
    s = lax.axis_index("s")
    sem_i = (sem_i0, sem_i1)
    sem_a = (sem_a0, sem_a1)
    sem_b = (sem_b0, sem_b1)
    pltpu.sync_copy(zeros32, acc.at[pl.ds(s * ROWS_T, ROWS_T)])
    plsc.subcore_barrier()

    base = s * (nchunks * CHUNK)

    def issue_idx(slot, cg):
        off = base + cg * CHUNK
        pltpu.async_copy(ni.at[pl.ds(off, CHUNK)], idx_n.at[slot], sem_i[slot])
        pltpu.async_copy(hi.at[pl.ds(off, CHUNK)], idx_h.at[slot], sem_i[slot])

    def wait_idx(slot):
        pltpu.make_async_copy(ni.at[pl.ds(0, CHUNK)], idx_n.at[slot],
                              sem_i[slot]).wait()
        pltpu.make_async_copy(hi.at[pl.ds(0, CHUNK)], idx_h.at[slot],
                              sem_i[slot]).wait()

    def transform_and_gather(slot):
        wait_idx(slot)
        for j in range(CHUNK // 16):
            vn = idx_n[slot, pl.ds(j * 16, 16)]
            sidx[slot, pl.ds(j * 16, 16)] = vn
            idxg_n[slot, pl.ds(j * 16, 16)] = vn + vn + c
            vh = idx_h[slot, pl.ds(j * 16, 16)]
            idxg_h[slot, pl.ds(j * 16, 16)] = vh + vh + c
        pltpu.async_copy(xt.at[idxg_n.at[slot]], bufa.at[slot], sem_a[slot])
        pltpu.async_copy(ht.at[idxg_h.at[slot]], bufb.at[slot], sem_b[slot])

    def compute_and_scatter(slot):
        pltpu.make_async_copy(xt.at[idxg_n.at[slot]], bufa.at[slot],
                              sem_a[slot]).wait()
        pltpu.make_async_copy(ht.at[idxg_h.at[slot]], bufb.at[slot],
                              sem_b[slot]).wait()

        @plsc.parallel_loop(0, CHUNK, step=1, unroll=8)
        def _(e):
            gf0 = bufa[slot, e, pl.ds(0, 16)] + bufb[slot, e, pl.ds(0, 16)]
            gf1 = bufa[slot, e, pl.ds(16, 16)] + bufb[slot, e, pl.ds(16, 16)]
            gc0 = bufa[slot, e, pl.ds(32, 16)] + bufb[slot, e, pl.ds(32, 16)]
            gc1 = bufa[slot, e, pl.ds(48, 16)] + bufb[slot, e, pl.ds(48, 16)]
            msg[e, pl.ds(0, 16)] = _gate(gf0, gc0)
            msg[e, pl.ds(16, 16)] = _gate(gf1, gc1)
        pltpu.sync_copy(msg, acc.at[sidx.at[slot]], add=True)

    issue_idx(0, 0)
    issue_idx(1, 1)
    transform_and_gather(0)

    def pair(p, _):
        for k in (0, 1):
            cg = 2 * p + k
            issue_idx(k, cg + 2)
            transform_and_gather(1 - k)
            compute_and_scatter(k)
        return 0

    lax.fori_loop(0, (nchunks - 1) // 2, pair, 0)
    compute_and_scatter(0)
    wait_idx(1)

    plsc.subcore_barrier()
    pltpu.sync_copy(acc.at[pl.ds(s * ROWS_T, ROWS_T)],
                    out.at[pl.ds(c * NPAD + s * ROWS_T, ROWS_T)])


@functools.cache
def _make_sc_conv(epad):
    nchunks = epad // NSUB // CHUNK
    return pl.kernel(
        functools.partial(_sc_conv_body, nchunks),
        out_type=jax.ShapeDtypeStruct((2 * NPAD, HH), jnp.float32),
        mesh=_mesh(),
        compiler_params=pltpu.CompilerParams(use_tc_tiling_on_sc=False),
        scratch_types=[
            pltpu.VMEM((2, CHUNK), jnp.int32),
            pltpu.VMEM((2, CHUNK), jnp.int32),
            pltpu.VMEM((2, CHUNK), jnp.int32),
            pltpu.VMEM((2, CHUNK), jnp.int32),
            pltpu.VMEM((2, CHUNK), jnp.int32),
            pltpu.VMEM((2, CHUNK, H), jnp.float32),
            pltpu.VMEM((2, CHUNK, H), jnp.float32),
            pltpu.VMEM((CHUNK, HH), jnp.float32),
            pltpu.VMEM_SHARED((NPAD, HH), jnp.float32),
            pltpu.SemaphoreType.DMA,
            pltpu.SemaphoreType.DMA,
            pltpu.SemaphoreType.DMA,
            pltpu.SemaphoreType.DMA,
            pltpu.SemaphoreType.DMA,
            pltpu.SemaphoreType.DMA,
        ],
    )


def _sc_cnt_body(nb, nm, bt, ones_hbm, zeros8, cb, cm, cg,
                 idx128, idx64, ones_v, spn, spg):
    c = lax.axis_index("c")
    s = lax.axis_index("s")
    pltpu.sync_copy(ones_hbm, ones_v)

    @pl.when(c == 0)
    def _():
        pltpu.sync_copy(zeros8, spn.at[pl.ds(s * ROWS_T, ROWS_T)])
        plsc.subcore_barrier()
        base = s * (EBPAD // NSUB)

        def chunk(g, _):
            pltpu.sync_copy(nb.at[pl.ds(base + g * CHUNK, CHUNK)], idx128)
            pltpu.sync_copy(ones_v, spn.at[idx128], add=True)
            return 0

        lax.fori_loop(0, EBPAD // NSUB // CHUNK, chunk, 0)
        plsc.subcore_barrier()
        pltpu.sync_copy(spn.at[pl.ds(s * ROWS_T, ROWS_T)],
                        cb.at[pl.ds(s * ROWS_T, ROWS_T)])

    @pl.when(c == 1)
    def _():
        pltpu.sync_copy(zeros8, spn.at[pl.ds(s * ROWS_T, ROWS_T)])
        plsc.subcore_barrier()
        base = s * (EMPAD // NSUB)

        def chunk(g, _):
            pltpu.sync_copy(nm.at[pl.ds(base + g * CHUNK, CHUNK)], idx128)
            pltpu.sync_copy(ones_v, spn.at[idx128], add=True)
            return 0

        lax.fori_loop(0, EMPAD // NSUB // CHUNK, chunk, 0)
        plsc.subcore_barrier()
        pltpu.sync_copy(spn.at[pl.ds(s * ROWS_T, ROWS_T)],
                        cm.at[pl.ds(s * ROWS_T, ROWS_T)])

        pltpu.sync_copy(zeros8.at[pl.ds(0, GROWS_T)],
                        spg.at[pl.ds(s * GROWS_T, GROWS_T)])
        plsc.subcore_barrier()
        gbase = s * ROWS_T

        def gchunk(g, _):
            pltpu.sync_copy(bt.at[pl.ds(gbase + g * 64, 64)], idx64)
            pltpu.sync_copy(ones_v.at[pl.ds(0, 64)], spg.at[idx64], add=True)
            return 0

        lax.fori_loop(0, ROWS_T // 64, gchunk, 0)
        plsc.subcore_barrier()
        pltpu.sync_copy(spg.at[pl.ds(s * GROWS_T, GROWS_T)],
                        cg.at[pl.ds(s * GROWS_T, GROWS_T)])


@functools.cache
def _get_sc_cnt():
    return pl.kernel(
        _sc_cnt_body,
        out_type=(jax.ShapeDtypeStruct((NPAD, 8), jnp.float32),
                  jax.ShapeDtypeStruct((NPAD, 8), jnp.float32),
                  jax.ShapeDtypeStruct((GPAD, 8), jnp.float32)),
        mesh=_mesh(),
        compiler_params=pltpu.CompilerParams(use_tc_tiling_on_sc=False),
        scratch_types=[
            pltpu.VMEM((CHUNK,), jnp.int32),
            pltpu.VMEM((64,), jnp.int32),
            pltpu.VMEM((CHUNK, 8), jnp.float32),
            pltpu.VMEM_SHARED((NPAD, 8), jnp.float32),
            pltpu.VMEM_SHARED((GPAD, 8), jnp.float32),
        ],
    )


def _sc_pool_body(xt, bt, zeros32, out, idx64, vbuf, spp):
    c = lax.axis_index("c")
    s = lax.axis_index("s")
    pltpu.sync_copy(zeros32.at[pl.ds(0, GROWS_T)],
                    spp.at[pl.ds(s * GROWS_T, GROWS_T)])
    plsc.subcore_barrier()
    base = s * ROWS_T

    def chunk(g, _):
        off = base + g * 64
        pltpu.sync_copy(bt.at[pl.ds(off, 64)], idx64)
        pltpu.sync_copy(xt.at[pl.ds(c * NPAD + off, 64)], vbuf)
        pltpu.sync_copy(vbuf, spp.at[idx64], add=True)
        return 0

    lax.fori_loop(0, ROWS_T // 64, chunk, 0)
    plsc.subcore_barrier()
    pltpu.sync_copy(spp.at[pl.ds(s * GROWS_T, GROWS_T)],
                    out.at[pl.ds(c * GPAD + s * GROWS_T, GROWS_T)])


@functools.cache
def _get_sc_pool():
    return pl.kernel(
        _sc_pool_body,
        out_type=jax.ShapeDtypeStruct((2 * GPAD, HH), jnp.float32),
        mesh=_mesh(),
        compiler_params=pltpu.CompilerParams(use_tc_tiling_on_sc=False),
        scratch_types=[
            pltpu.VMEM((64,), jnp.int32),
            pltpu.VMEM((64, HH), jnp.float32),
            pltpu.VMEM_SHARED((GPAD, HH), jnp.float32),
        ],
    )


def _embed_atom_body(a_ref, w_ref, b_ref, wt_ref, bt_ref, x_ref, t_ref):
    x = jnp.dot(a_ref[...], w_ref[...],
                preferred_element_type=jnp.float32) + b_ref[...]
    x_ref[...] = x
    p = jnp.dot(x, wt_ref[...], preferred_element_type=jnp.float32) + bt_ref[...]
    t_ref[...] = p.reshape(p.shape[0], 2, H)


def _embed_hedge_body(a_ref, w_ref, b_ref, wb_ref, t_ref):
    h = jnp.dot(a_ref[...], w_ref[...],
                preferred_element_type=jnp.float32) + b_ref[...]
    p = jnp.dot(h, wb_ref[...], preferred_element_type=jnp.float32)
    t_ref[...] = p.reshape(p.shape[0], 2, H)


def _update_body(x_ref, a0_ref, a1_ref, cnt_ref, wt_ref, bt_ref,
                 x_out, t_ref):
    aggr = jnp.concatenate([a0_ref[...], a1_ref[...]], axis=1)
    cnt = jnp.maximum(cnt_ref[:, 0:1], 1.0)
    xn = jax.nn.softplus(x_ref[...] + aggr / cnt)
    x_out[...] = xn
    p = jnp.dot(xn, wt_ref[...], preferred_element_type=jnp.float32) + bt_ref[...]
    t_ref[...] = p.reshape(p.shape[0], 2, H)


def _update_last_body(x_ref, a0_ref, a1_ref, cnt_ref, t_ref):
    aggr = jnp.concatenate([a0_ref[...], a1_ref[...]], axis=1)
    cnt = jnp.maximum(cnt_ref[:, 0:1], 1.0)
    xn = jax.nn.softplus(x_ref[...] + aggr / cnt)
    t_ref[0, :, :] = xn[:, 0:HH]
    t_ref[1, :, :] = xn[:, HH:H]


def _head_body(p0_ref, p1_ref, cg_ref, w1, b1, w2, b2, w3, b3, wo, bo, o_ref):
    s = jnp.concatenate([p0_ref[...], p1_ref[...]], axis=1)
    cnt = jnp.maximum(cg_ref[:, 0:1], 1.0)
    h = s / cnt
    h = jax.nn.softplus(jnp.dot(h, w1[...], preferred_element_type=jnp.float32) + b1[...])
    h = jax.nn.softplus(jnp.dot(h, w2[...], preferred_element_type=jnp.float32) + b2[...])
    h = jax.nn.softplus(jnp.dot(h, w3[...], preferred_element_type=jnp.float32) + b3[...])
    o_ref[...] = jnp.dot(h, wo[...], preferred_element_type=jnp.float32) + bo[...]


def _full(shape):
    return pl.BlockSpec(shape, lambda i: tuple(0 for _ in shape))


_BLK = 512


def _embed_atom(attrs, w, b, wt, bt):
    n = attrs.shape[0]
    return pl.pallas_call(
        _embed_atom_body,
        grid=(n // _BLK,),
        in_specs=[
            pl.BlockSpec((_BLK, attrs.shape[1]), lambda i: (i, 0)),
            _full(w.shape), _full(b.shape), _full(wt.shape), _full(bt.shape),
        ],
        out_specs=[
            pl.BlockSpec((_BLK, H), lambda i: (i, 0)),
            pl.BlockSpec((_BLK, 2, H), lambda i: (i, 0, 0)),
        ],
        out_shape=[
            jax.ShapeDtypeStruct((n, H), jnp.float32),
            jax.ShapeDtypeStruct((n, 2, H), jnp.float32),
        ],
    )(attrs, w, b, wt, bt)


def _embed_hedge(attrs, w, b, wb):
    n = attrs.shape[0]
    return pl.pallas_call(
        _embed_hedge_body,
        grid=(n // _BLK,),
        in_specs=[
            pl.BlockSpec((_BLK, attrs.shape[1]), lambda i: (i, 0)),
            _full(w.shape), _full(b.shape), _full(wb.shape),
        ],
        out_specs=pl.BlockSpec((_BLK, 2, H), lambda i: (i, 0, 0)),
        out_shape=jax.ShapeDtypeStruct((n, 2, H), jnp.float32),
    )(attrs, w, b, wb)


def _update(x, aggr, cnt, wt, bt):
    nb = NPAD // _BLK
    return pl.pallas_call(
        _update_body,
        grid=(nb,),
        in_specs=[
            pl.BlockSpec((_BLK, H), lambda i: (i, 0)),
            pl.BlockSpec((_BLK, HH), lambda i: (i, 0)),
            pl.BlockSpec((_BLK, HH), lambda i, _n=nb: (i + _n, 0)),
            pl.BlockSpec((_BLK, 8), lambda i: (i, 0)),
            _full(wt.shape), _full(bt.shape),
        ],
        out_specs=[
            pl.BlockSpec((_BLK, H), lambda i: (i, 0)),
            pl.BlockSpec((_BLK, 2, H), lambda i: (i, 0, 0)),
        ],
        out_shape=[
            jax.ShapeDtypeStruct((NPAD, H), jnp.float32),
            jax.ShapeDtypeStruct((NPAD, 2, H), jnp.float32),
        ],
    )(x, aggr, aggr, cnt, wt, bt)


def _update_last(x, aggr, cnt):
    nb = NPAD // _BLK
    return pl.pallas_call(
        _update_last_body,
        grid=(nb,),
        in_specs=[
            pl.BlockSpec((_BLK, H), lambda i: (i, 0)),
            pl.BlockSpec((_BLK, HH), lambda i: (i, 0)),
            pl.BlockSpec((_BLK, HH), lambda i, _n=nb: (i + _n, 0)),
            pl.BlockSpec((_BLK, 8), lambda i: (i, 0)),
        ],
        out_specs=pl.BlockSpec((2, _BLK, HH), lambda i: (0, i, 0)),
        out_shape=jax.ShapeDtypeStruct((2, NPAD, HH), jnp.float32),
    )(x, aggr, aggr, cnt)


def _head(p0, p1, cg, w1, b1, w2, b2, w3, b3, wo, bo):
    return pl.pallas_call(
        _head_body,
        grid=(1,),
        in_specs=[
            pl.BlockSpec((NGRAPH, HH), lambda i: (0, 0)),
            pl.BlockSpec((NGRAPH, HH), lambda i: (0, 0)),
            pl.BlockSpec((NGRAPH, 8), lambda i: (0, 0)),
            _full(w1.shape), _full(b1.shape), _full(w2.shape), _full(b2.shape),
            _full(w3.shape), _full(b3.shape), _full(wo.shape), _full(bo.shape),
        ],
        out_specs=pl.BlockSpec((NGRAPH, 1), lambda i: (0, 0)),
        out_shape=jax.ShapeDtypeStruct((NGRAPH, 1), jnp.float32),
    )(p0, p1, cg, w1, b1, w2, b2, w3, b3, wo, bo)


def _pad_rows(a, n, val=0):
    return jnp.pad(a, ((0, n - a.shape[0]),) + ((0, 0),) * (a.ndim - 1),
                   constant_values=val)


def kernel(atom_attrs, bond_attrs, motif_attrs, bond_index, motif_index,
           batch, W_embed, b_embed, W_bembed, b_bembed, W_membed, b_membed,
           conv_Wf, conv_bf, conv_Wc, conv_bc,
           W1, b1, W2, b2, W3, b3, Wout, bout):
    f32 = jnp.float32
    aa = jnp.pad(atom_attrs, ((0, NPAD - NATOM), (0, 4)))
    ma = jnp.pad(motif_attrs, ((0, NMPAD - NMOTIF), (0, 2)))
    ba = _pad_rows(bond_attrs, NBPAD)
    we = jnp.pad(W_embed, ((0, 4), (0, 0)))
    wm = jnp.pad(W_membed, ((0, 2), (0, 0)))

    hi_b = _pad_rows(bond_index[0], EBPAD + IDXPAD, NBOND)
    ni_b = _pad_rows(bond_index[1], EBPAD + IDXPAD, NATOM)
    hi_m = _pad_rows(motif_index[0], EMPAD + IDXPAD, NMOTIF)
    ni_m = _pad_rows(motif_index[1], EMPAD + IDXPAD, NATOM)
    bt = _pad_rows(batch, NPAD, NGRAPH)

    perm = jnp.concatenate([jnp.arange(HH), jnp.arange(H, H + HH),
                            jnp.arange(HH, H), jnp.arange(H + HH, 2 * H)])
    wtop = [jnp.concatenate([conv_Wf[i][:H], conv_Wc[i][:H]], axis=1)[:, perm]
            for i in range(6)]
    btop = [jnp.concatenate([conv_bf[i], conv_bc[i]])[perm] for i in range(6)]
    wbot = [jnp.concatenate([conv_Wf[i][H:], conv_Wc[i][H:]], axis=1)[:, perm]
            for i in range(6)]

    zeros32 = jnp.zeros((ROWS_T, HH), f32)
    zeros8 = jnp.zeros((ROWS_T, 8), f32)
    ones8 = jnp.ones((CHUNK, 8), f32)

    cb, cm, cg = _get_sc_cnt()(ni_b, ni_m, bt, ones8, zeros8)
    x, t = _embed_atom(aa, we, b_embed.reshape(1, -1),
                       wtop[0], btop[0].reshape(1, -1))

    conv_b = _make_sc_conv(EBPAD)
    conv_m = _make_sc_conv(EMPAD)

    for l in range(6):
        if l % 2 == 0:
            ht = _embed_hedge(ba, W_bembed, b_bembed.reshape(1, -1), wbot[l])
            aggr = conv_b(t.reshape(2 * NPAD, H), ht.reshape(2 * NBPAD, H),
                          ni_b, hi_b, zeros32)
            cnt = cb
        else:
            ht = _embed_hedge(ma, wm, b_membed.reshape(1, -1), wbot[l])
            aggr = conv_m(t.reshape(2 * NPAD, H), ht.reshape(2 * NMPAD, H),
                          ni_m, hi_m, zeros32)
            cnt = cm
        if l < 5:
            x, t = _update(x, aggr, cnt, wtop[l + 1],
                           btop[l + 1].reshape(1, -1))
        else:
            xt = _update_last(x, aggr, cnt)

    pooled = _get_sc_pool()(xt.reshape(2 * NPAD, HH), bt, zeros32)
    return _head(pooled[:NGRAPH], pooled[GPAD:GPAD + NGRAPH], cg[:NGRAPH],
                 W1, b1.reshape(1, -1), W2, b2.reshape(1, -1),
                 W3, b3.reshape(1, -1), Wout, bout.reshape(1, -1))


# speedup vs baseline: 1.1530x; 1.1530x over previous
import functools

import jax
import jax.numpy as jnp
from jax import lax
from jax.experimental import pallas as pl
from jax.experimental.pallas import tpu as pltpu
from jax.experimental.pallas import tpu_sc as plsc

H = 64
HH = 32
NATOM = 50000
NPAD = 50176
NBOND = 400000
NBPAD = 400384
NMOTIF = 40000
NMPAD = 40448
EBPAD = 803328
EMPAD = 400896
IDXPAD = 96
NGRAPH = 1024
GPAD = 1152
NSUB = 16
ROWS_T = NPAD // NSUB
GROWS_T = GPAD // NSUB
CHUNK = 96

_LOG1P = (
    3.5075520539162852e-06, 0.9997924357286234, -0.4969779111677995,
    0.31459053537163134, -0.18878267362227552, 0.0817268083763448,
    -0.017208061121549833,
)
_NDEG = len(_LOG1P) - 1

@functools.cache
def _mesh():
    return plsc.VectorSubcoreMesh(core_axis_name="c", subcore_axis_name="s",
                                  num_cores=2, num_subcores=NSUB)


def _gate(a, b):
    sig = 1.0 / (1.0 + jnp.exp(-a))
    t = jnp.exp(-jnp.abs(b))
    p = jnp.full_like(t, _LOG1P[_NDEG])
    for k in range(_NDEG - 1, -1, -1):
        p = p * t + _LOG1P[k]
    sp = jnp.maximum(b, 0.0) + p
    return sig * sp


def _sc_conv_body(nchunks, xt, ht, ni, hi, zeros32, out,
                  idx_n, idx_h, idxg_n, idxg_h, sidx, bufa, bufb, msg,
                  acc, sem_i0, sem_i1, sem_a0, sem_a1, sem_b0, sem_b1):
    c = lax.axis_index("c")
    s = lax.axis_index("s")
    sem_i = (sem_i0, sem_i1)
    sem_a = (sem_a0, sem_a1)
    sem_b = (sem_b0, sem_b1)
    pltpu.sync_copy(zeros32, acc.at[pl.ds(s * ROWS_T, ROWS_T)])
    plsc.subcore_barrier()

    base = s * (nchunks * CHUNK)

    def issue_idx(slot, cg):
        off = base + cg * CHUNK
        pltpu.async_copy(ni.at[pl.ds(off, CHUNK)], idx_n.at[slot], sem_i[slot])
        pltpu.async_copy(hi.at[pl.ds(off, CHUNK)], idx_h.at[slot], sem_i[slot])

    def wait_idx(slot):
        pltpu.make_async_copy(ni.at[pl.ds(0, CHUNK)], idx_n.at[slot],
                              sem_i[slot]).wait()
        pltpu.make_async_copy(hi.at[pl.ds(0, CHUNK)], idx_h.at[slot],
                              sem_i[slot]).wait()

    def transform_and_gather(slot):
        wait_idx(slot)
        for j in range(CHUNK // 16):
            vn = idx_n[slot, pl.ds(j * 16, 16)]
            sidx[slot, pl.ds(j * 16, 16)] = vn
            idxg_n[slot, pl.ds(j * 16, 16)] = vn + vn + c
            vh = idx_h[slot, pl.ds(j * 16, 16)]
            idxg_h[slot, pl.ds(j * 16, 16)] = vh + vh + c
        pltpu.async_copy(xt.at[idxg_n.at[slot]], bufa.at[slot], sem_a[slot])
        pltpu.async_copy(ht.at[idxg_h.at[slot]], bufb.at[slot], sem_b[slot])

    def compute_and_scatter(slot):
        pltpu.make_async_copy(xt.at[idxg_n.at[slot]], bufa.at[slot],
                              sem_a[slot]).wait()
        pltpu.make_async_copy(ht.at[idxg_h.at[slot]], bufb.at[slot],
                              sem_b[slot]).wait()

        @plsc.parallel_loop(0, CHUNK, step=1, unroll=8)
        def _(e):
            gf0 = bufa[slot, e, pl.ds(0, 16)] + bufb[slot, e, pl.ds(0, 16)]
            gf1 = bufa[slot, e, pl.ds(16, 16)] + bufb[slot, e, pl.ds(16, 16)]
            gc0 = bufa[slot, e, pl.ds(32, 16)] + bufb[slot, e, pl.ds(32, 16)]
            gc1 = bufa[slot, e, pl.ds(48, 16)] + bufb[slot, e, pl.ds(48, 16)]
            msg[e, pl.ds(0, 16)] = _gate(gf0, gc0)
            msg[e, pl.ds(16, 16)] = _gate(gf1, gc1)
        pltpu.sync_copy(msg, acc.at[sidx.at[slot]], add=True)

    issue_idx(0, 0)
    issue_idx(1, 1)
    transform_and_gather(0)

    def pair(p, _):
        for k in (0, 1):
            cg = 2 * p + k
            issue_idx(k, cg + 2)
            transform_and_gather(1 - k)
            compute_and_scatter(k)
        return 0

    lax.fori_loop(0, (nchunks - 1) // 2, pair, 0)
    compute_and_scatter(0)
    wait_idx(1)

    plsc.subcore_barrier()
    pltpu.sync_copy(acc.at[pl.ds(s * ROWS_T, ROWS_T)],
                    out.at[pl.ds(c * NPAD + s * ROWS_T, ROWS_T)])


@functools.cache
def _make_sc_conv(epad):
    nchunks = epad // NSUB // CHUNK
    return pl.kernel(
        functools.partial(_sc_conv_body, nchunks),
        out_type=jax.ShapeDtypeStruct((2 * NPAD, HH), jnp.float32),
        mesh=_mesh(),
        compiler_params=pltpu.CompilerParams(use_tc_tiling_on_sc=False),
        scratch_types=[
            pltpu.VMEM((2, CHUNK), jnp.int32),
            pltpu.VMEM((2, CHUNK), jnp.int32),
            pltpu.VMEM((2, CHUNK), jnp.int32),
            pltpu.VMEM((2, CHUNK), jnp.int32),
            pltpu.VMEM((2, CHUNK), jnp.int32),
            pltpu.VMEM((2, CHUNK, H), jnp.float32),
            pltpu.VMEM((2, CHUNK, H), jnp.float32),
            pltpu.VMEM((CHUNK, HH), jnp.float32),
            pltpu.VMEM_SHARED((NPAD, HH), jnp.float32),
            pltpu.SemaphoreType.DMA,
            pltpu.SemaphoreType.DMA,
            pltpu.SemaphoreType.DMA,
            pltpu.SemaphoreType.DMA,
            pltpu.SemaphoreType.DMA,
            pltpu.SemaphoreType.DMA,
        ],
    )


def _sc_cnt_body(nb, nm, bt, ones_hbm, zeros8, cb, cm, cg,
                 idx128, idx64, ones_v, spn, spg):
    c = lax.axis_index("c")
    s = lax.axis_index("s")
    pltpu.sync_copy(ones_hbm, ones_v)

    @pl.when(c == 0)
    def _():
        pltpu.sync_copy(zeros8, spn.at[pl.ds(s * ROWS_T, ROWS_T)])
        plsc.subcore_barrier()
        base = s * (EBPAD // NSUB)

        def chunk(g, _):
            pltpu.sync_copy(nb.at[pl.ds(base + g * CHUNK, CHUNK)], idx128)
            pltpu.sync_copy(ones_v, spn.at[idx128], add=True)
            return 0

        lax.fori_loop(0, EBPAD // NSUB // CHUNK, chunk, 0)
        plsc.subcore_barrier()
        pltpu.sync_copy(spn.at[pl.ds(s * ROWS_T, ROWS_T)],
                        cb.at[pl.ds(s * ROWS_T, ROWS_T)])

    @pl.when(c == 1)
    def _():
        pltpu.sync_copy(zeros8, spn.at[pl.ds(s * ROWS_T, ROWS_T)])
        plsc.subcore_barrier()
        base = s * (EMPAD // NSUB)

        def chunk(g, _):
            pltpu.sync_copy(nm.at[pl.ds(base + g * CHUNK, CHUNK)], idx128)
            pltpu.sync_copy(ones_v, spn.at[idx128], add=True)
            return 0

        lax.fori_loop(0, EMPAD // NSUB // CHUNK, chunk, 0)
        plsc.subcore_barrier()
        pltpu.sync_copy(spn.at[pl.ds(s * ROWS_T, ROWS_T)],
                        cm.at[pl.ds(s * ROWS_T, ROWS_T)])

        pltpu.sync_copy(zeros8.at[pl.ds(0, GROWS_T)],
                        spg.at[pl.ds(s * GROWS_T, GROWS_T)])
        plsc.subcore_barrier()
        gbase = s * ROWS_T

        def gchunk(g, _):
            pltpu.sync_copy(bt.at[pl.ds(gbase + g * 64, 64)], idx64)
            pltpu.sync_copy(ones_v.at[pl.ds(0, 64)], spg.at[idx64], add=True)
            return 0

        lax.fori_loop(0, ROWS_T // 64, gchunk, 0)
        plsc.subcore_barrier()
        pltpu.sync_copy(spg.at[pl.ds(s * GROWS_T, GROWS_T)],
                        cg.at[pl.ds(s * GROWS_T, GROWS_T)])


@functools.cache
def _get_sc_cnt():
    return pl.kernel(
        _sc_cnt_body,
        out_type=(jax.ShapeDtypeStruct((NPAD, 8), jnp.float32),
                  jax.ShapeDtypeStruct((NPAD, 8), jnp.float32),
                  jax.ShapeDtypeStruct((GPAD, 8), jnp.float32)),
        mesh=_mesh(),
        compiler_params=pltpu.CompilerParams(use_tc_tiling_on_sc=False),
        scratch_types=[
            pltpu.VMEM((CHUNK,), jnp.int32),
            pltpu.VMEM((64,), jnp.int32),
            pltpu.VMEM((CHUNK, 8), jnp.float32),
            pltpu.VMEM_SHARED((NPAD, 8), jnp.float32),
            pltpu.VMEM_SHARED((GPAD, 8), jnp.float32),
        ],
    )


def _sc_pool_body(xt, bt, zeros32, out, idx64, vbuf, spp):
    c = lax.axis_index("c")
    s = lax.axis_index("s")
    pltpu.sync_copy(zeros32.at[pl.ds(0, GROWS_T)],
                    spp.at[pl.ds(s * GROWS_T, GROWS_T)])
    plsc.subcore_barrier()
    base = s * ROWS_T

    def chunk(g, _):
        off = base + g * 64
        pltpu.sync_copy(bt.at[pl.ds(off, 64)], idx64)
        pltpu.sync_copy(xt.at[pl.ds(c * NPAD + off, 64)], vbuf)
        pltpu.sync_copy(vbuf, spp.at[idx64], add=True)
        return 0

    lax.fori_loop(0, ROWS_T // 64, chunk, 0)
    plsc.subcore_barrier()
    pltpu.sync_copy(spp.at[pl.ds(s * GROWS_T, GROWS_T)],
                    out.at[pl.ds(c * GPAD + s * GROWS_T, GROWS_T)])


@functools.cache
def _get_sc_pool():
    return pl.kernel(
        _sc_pool_body,
        out_type=jax.ShapeDtypeStruct((2 * GPAD, HH), jnp.float32),
        mesh=_mesh(),
        compiler_params=pltpu.CompilerParams(use_tc_tiling_on_sc=False),
        scratch_types=[
            pltpu.VMEM((64,), jnp.int32),
            pltpu.VMEM((64, HH), jnp.float32),
            pltpu.VMEM_SHARED((GPAD, HH), jnp.float32),
        ],
    )


def _embed_atom_body(a_ref, w_ref, b_ref, wt_ref, bt_ref, x_ref, t_ref):
    x = jnp.dot(a_ref[...], w_ref[...],
                preferred_element_type=jnp.float32) + b_ref[...]
    x_ref[...] = x
    p = jnp.dot(x, wt_ref[...], preferred_element_type=jnp.float32) + bt_ref[...]
    t_ref[...] = p.reshape(p.shape[0], 2, H)


def _embed_hedge_body(a_ref, w_ref, b_ref, wb_ref, t0_ref, t1_ref, t2_ref):
    h = jnp.dot(a_ref[...], w_ref[...],
                preferred_element_type=jnp.float32) + b_ref[...]
    for j, t_ref in enumerate((t0_ref, t1_ref, t2_ref)):
        p = jnp.dot(h, wb_ref[j], preferred_element_type=jnp.float32)
        t_ref[...] = p.reshape(p.shape[0], 2, H)


def _update_body(x_ref, a0_ref, a1_ref, cnt_ref, wt_ref, bt_ref,
                 x_out, t_ref):
    aggr = jnp.concatenate([a0_ref[...], a1_ref[...]], axis=1)
    cnt = jnp.maximum(cnt_ref[:, 0:1], 1.0)
    xn = jax.nn.softplus(x_ref[...] + aggr / cnt)
    x_out[...] = xn
    p = jnp.dot(xn, wt_ref[...], preferred_element_type=jnp.float32) + bt_ref[...]
    t_ref[...] = p.reshape(p.shape[0], 2, H)


def _update_last_body(x_ref, a0_ref, a1_ref, cnt_ref, t_ref):
    aggr = jnp.concatenate([a0_ref[...], a1_ref[...]], axis=1)
    cnt = jnp.maximum(cnt_ref[:, 0:1], 1.0)
    xn = jax.nn.softplus(x_ref[...] + aggr / cnt)
    t_ref[0, :, :] = xn[:, 0:HH]
    t_ref[1, :, :] = xn[:, HH:H]


def _head_body(p0_ref, p1_ref, cg_ref, w1, b1, w2, b2, w3, b3, wo, bo, o_ref):
    s = jnp.concatenate([p0_ref[...], p1_ref[...]], axis=1)
    cnt = jnp.maximum(cg_ref[:, 0:1], 1.0)
    h = s / cnt
    h = jax.nn.softplus(jnp.dot(h, w1[...], preferred_element_type=jnp.float32) + b1[...])
    h = jax.nn.softplus(jnp.dot(h, w2[...], preferred_element_type=jnp.float32) + b2[...])
    h = jax.nn.softplus(jnp.dot(h, w3[...], preferred_element_type=jnp.float32) + b3[...])
    o_ref[...] = jnp.dot(h, wo[...], preferred_element_type=jnp.float32) + bo[...]


def _full(shape):
    return pl.BlockSpec(shape, lambda i: tuple(0 for _ in shape))


_BLK = 512


def _embed_atom(attrs, w, b, wt, bt):
    n = attrs.shape[0]
    return pl.pallas_call(
        _embed_atom_body,
        grid=(n // _BLK,),
        in_specs=[
            pl.BlockSpec((_BLK, attrs.shape[1]), lambda i: (i, 0)),
            _full(w.shape), _full(b.shape), _full(wt.shape), _full(bt.shape),
        ],
        out_specs=[
            pl.BlockSpec((_BLK, H), lambda i: (i, 0)),
            pl.BlockSpec((_BLK, 2, H), lambda i: (i, 0, 0)),
        ],
        out_shape=[
            jax.ShapeDtypeStruct((n, H), jnp.float32),
            jax.ShapeDtypeStruct((n, 2, H), jnp.float32),
        ],
    )(attrs, w, b, wt, bt)


def _embed_hedge(attrs, w, b, wb):
    n = attrs.shape[0]
    ts = jax.ShapeDtypeStruct((n, 2, H), jnp.float32)
    return pl.pallas_call(
        _embed_hedge_body,
        grid=(n // _BLK,),
        in_specs=[
            pl.BlockSpec((_BLK, attrs.shape[1]), lambda i: (i, 0)),
            _full(w.shape), _full(b.shape), _full(wb.shape),
        ],
        out_specs=[pl.BlockSpec((_BLK, 2, H), lambda i: (i, 0, 0))] * 3,
        out_shape=[ts, ts, ts],
    )(attrs, w, b, wb)


def _update(x, aggr, cnt, wt, bt):
    nb = NPAD // _BLK
    return pl.pallas_call(
        _update_body,
        grid=(nb,),
        in_specs=[
            pl.BlockSpec((_BLK, H), lambda i: (i, 0)),
            pl.BlockSpec((_BLK, HH), lambda i: (i, 0)),
            pl.BlockSpec((_BLK, HH), lambda i, _n=nb: (i + _n, 0)),
            pl.BlockSpec((_BLK, 8), lambda i: (i, 0)),
            _full(wt.shape), _full(bt.shape),
        ],
        out_specs=[
            pl.BlockSpec((_BLK, H), lambda i: (i, 0)),
            pl.BlockSpec((_BLK, 2, H), lambda i: (i, 0, 0)),
        ],
        out_shape=[
            jax.ShapeDtypeStruct((NPAD, H), jnp.float32),
            jax.ShapeDtypeStruct((NPAD, 2, H), jnp.float32),
        ],
    )(x, aggr, aggr, cnt, wt, bt)


def _update_last(x, aggr, cnt):
    nb = NPAD // _BLK
    return pl.pallas_call(
        _update_last_body,
        grid=(nb,),
        in_specs=[
            pl.BlockSpec((_BLK, H), lambda i: (i, 0)),
            pl.BlockSpec((_BLK, HH), lambda i: (i, 0)),
            pl.BlockSpec((_BLK, HH), lambda i, _n=nb: (i + _n, 0)),
            pl.BlockSpec((_BLK, 8), lambda i: (i, 0)),
        ],
        out_specs=pl.BlockSpec((2, _BLK, HH), lambda i: (0, i, 0)),
        out_shape=jax.ShapeDtypeStruct((2, NPAD, HH), jnp.float32),
    )(x, aggr, aggr, cnt)


def _head(p0, p1, cg, w1, b1, w2, b2, w3, b3, wo, bo):
    return pl.pallas_call(
        _head_body,
        grid=(1,),
        in_specs=[
            pl.BlockSpec((NGRAPH, HH), lambda i: (0, 0)),
            pl.BlockSpec((NGRAPH, HH), lambda i: (0, 0)),
            pl.BlockSpec((NGRAPH, 8), lambda i: (0, 0)),
            _full(w1.shape), _full(b1.shape), _full(w2.shape), _full(b2.shape),
            _full(w3.shape), _full(b3.shape), _full(wo.shape), _full(bo.shape),
        ],
        out_specs=pl.BlockSpec((NGRAPH, 1), lambda i: (0, 0)),
        out_shape=jax.ShapeDtypeStruct((NGRAPH, 1), jnp.float32),
    )(p0, p1, cg, w1, b1, w2, b2, w3, b3, wo, bo)


def _pad_rows(a, n, val=0):
    return jnp.pad(a, ((0, n - a.shape[0]),) + ((0, 0),) * (a.ndim - 1),
                   constant_values=val)


def kernel(atom_attrs, bond_attrs, motif_attrs, bond_index, motif_index,
           batch, W_embed, b_embed, W_bembed, b_bembed, W_membed, b_membed,
           conv_Wf, conv_bf, conv_Wc, conv_bc,
           W1, b1, W2, b2, W3, b3, Wout, bout):
    f32 = jnp.float32
    aa = jnp.pad(atom_attrs, ((0, NPAD - NATOM), (0, 4)))
    ma = jnp.pad(motif_attrs, ((0, NMPAD - NMOTIF), (0, 2)))
    ba = _pad_rows(bond_attrs, NBPAD)
    we = jnp.pad(W_embed, ((0, 4), (0, 0)))
    wm = jnp.pad(W_membed, ((0, 2), (0, 0)))

    hi_b = _pad_rows(bond_index[0], EBPAD + IDXPAD, NBOND)
    ni_b = _pad_rows(bond_index[1], EBPAD + IDXPAD, NATOM)
    hi_m = _pad_rows(motif_index[0], EMPAD + IDXPAD, NMOTIF)
    ni_m = _pad_rows(motif_index[1], EMPAD + IDXPAD, NATOM)
    bt = _pad_rows(batch, NPAD, NGRAPH)

    perm = jnp.concatenate([jnp.arange(HH), jnp.arange(H, H + HH),
                            jnp.arange(HH, H), jnp.arange(H + HH, 2 * H)])
    wtop = [jnp.concatenate([conv_Wf[i][:H], conv_Wc[i][:H]], axis=1)[:, perm]
            for i in range(6)]
    btop = [jnp.concatenate([conv_bf[i], conv_bc[i]])[perm] for i in range(6)]
    wbot = [jnp.concatenate([conv_Wf[i][H:], conv_Wc[i][H:]], axis=1)[:, perm]
            for i in range(6)]

    zeros32 = jnp.zeros((ROWS_T, HH), f32)
    zeros8 = jnp.zeros((ROWS_T, 8), f32)
    ones8 = jnp.ones((CHUNK, 8), f32)

    cb, cm, cg = _get_sc_cnt()(ni_b, ni_m, bt, ones8, zeros8)
    x, t = _embed_atom(aa, we, b_embed.reshape(1, -1),
                       wtop[0], btop[0].reshape(1, -1))

    conv_b = _make_sc_conv(EBPAD)
    conv_m = _make_sc_conv(EMPAD)

    tb = _embed_hedge(ba, W_bembed, b_bembed.reshape(1, -1),
                      jnp.stack([wbot[i] for i in (0, 2, 4)]))
    tm = _embed_hedge(ma, wm, b_membed.reshape(1, -1),
                      jnp.stack([wbot[i] for i in (1, 3, 5)]))

    for l in range(6):
        if l % 2 == 0:
            ht = tb[l // 2].reshape(2 * NBPAD, H)
            aggr = conv_b(t.reshape(2 * NPAD, H), ht, ni_b, hi_b, zeros32)
            cnt = cb
        else:
            ht = tm[l // 2].reshape(2 * NMPAD, H)
            aggr = conv_m(t.reshape(2 * NPAD, H), ht, ni_m, hi_m, zeros32)
            cnt = cm
        if l < 5:
            x, t = _update(x, aggr, cnt, wtop[l + 1],
                           btop[l + 1].reshape(1, -1))
        else:
            xt = _update_last(x, aggr, cnt)

    pooled = _get_sc_pool()(xt.reshape(2 * NPAD, HH), bt, zeros32)
    return _head(pooled[:NGRAPH], pooled[GPAD:GPAD + NGRAPH], cg[:NGRAPH],
                 W1, b1.reshape(1, -1), W2, b2.reshape(1, -1),
                 W3, b3.reshape(1, -1), Wout, bout.reshape(1, -1))


# speedup vs baseline: 1.1530x; 1.0000x over previous
import functools

import jax
import jax.numpy as jnp
from jax import lax
from jax.experimental import pallas as pl
from jax.experimental.pallas import tpu as pltpu
from jax.experimental.pallas import tpu_sc as plsc

H = 64
HH = 32
NATOM = 50000
NPAD = 50176
NBOND = 400000
NBPAD = 400384
NMOTIF = 40000
NMPAD = 40448
EBPAD = 803328
EMPAD = 400896
IDXPAD = 96
NGRAPH = 1024
GPAD = 1152
NSUB = 16
ROWS_T = NPAD // NSUB
GROWS_T = GPAD // NSUB
CHUNK = 96

_LOG1P = (
    3.5075520539162852e-06, 0.9997924357286234, -0.4969779111677995,
    0.31459053537163134, -0.18878267362227552, 0.0817268083763448,
    -0.017208061121549833,
)
_NDEG = len(_LOG1P) - 1

@functools.cache
def _mesh():
    return plsc.VectorSubcoreMesh(core_axis_name="c", subcore_axis_name="s",
                                  num_cores=2, num_subcores=NSUB)


def _gate(a, b):
    sig = 1.0 / (1.0 + jnp.exp(-a))
    t = jnp.exp(-jnp.abs(b))
    p = jnp.full_like(t, _LOG1P[_NDEG])
    for k in range(_NDEG - 1, -1, -1):
        p = p * t + _LOG1P[k]
    sp = jnp.maximum(b, 0.0) + p
    return sig * sp


def _sc_conv_body(nchunks, xt, ht, ni, hi, zeros32, out,
                  idx_n, idx_h, idxg_n, idxg_h, sidx, bufa, bufb, msg,
                  acc, sem_i0, sem_i1, sem_a0, sem_a1, sem_b0, sem_b1):
    c = lax.axis_index("c")
    s = lax.axis_index("s")
    sem_i = (sem_i0, sem_i1)
    sem_a = (sem_a0, sem_a1)
    sem_b = (sem_b0, sem_b1)
    pltpu.sync_copy(zeros32, acc.at[pl.ds(s * ROWS_T, ROWS_T)])
    plsc.subcore_barrier()

    base = s * (nchunks * CHUNK)

    def issue_idx(slot, cg):
        off = base + cg * CHUNK
        pltpu.async_copy(ni.at[pl.ds(off, CHUNK)], idx_n.at[slot], sem_i[slot])
        pltpu.async_copy(hi.at[pl.ds(off, CHUNK)], idx_h.at[slot], sem_i[slot])

    def wait_idx(slot):
        pltpu.make_async_copy(ni.at[pl.ds(0, CHUNK)], idx_n.at[slot],
                              sem_i[slot]).wait()
        pltpu.make_async_copy(hi.at[pl.ds(0, CHUNK)], idx_h.at[slot],
                              sem_i[slot]).wait()

    def transform_and_gather(slot):
        wait_idx(slot)
        for j in range(CHUNK // 16):
            vn = idx_n[slot, pl.ds(j * 16, 16)]
            sidx[slot, pl.ds(j * 16, 16)] = vn
            idxg_n[slot, pl.ds(j * 16, 16)] = vn + vn + c
            vh = idx_h[slot, pl.ds(j * 16, 16)]
            idxg_h[slot, pl.ds(j * 16, 16)] = vh + vh + c
        pltpu.async_copy(xt.at[idxg_n.at[slot]], bufa.at[slot], sem_a[slot])
        pltpu.async_copy(ht.at[idxg_h.at[slot]], bufb.at[slot], sem_b[slot])

    def compute_and_scatter(slot):
        pltpu.make_async_copy(xt.at[idxg_n.at[slot]], bufa.at[slot],
                              sem_a[slot]).wait()
        pltpu.make_async_copy(ht.at[idxg_h.at[slot]], bufb.at[slot],
                              sem_b[slot]).wait()

        @plsc.parallel_loop(0, CHUNK, step=1, unroll=8)
        def _(e):
            gf0 = bufa[slot, e, pl.ds(0, 16)] + bufb[slot, e, pl.ds(0, 16)]
            gf1 = bufa[slot, e, pl.ds(16, 16)] + bufb[slot, e, pl.ds(16, 16)]
            gc0 = bufa[slot, e, pl.ds(32, 16)] + bufb[slot, e, pl.ds(32, 16)]
            gc1 = bufa[slot, e, pl.ds(48, 16)] + bufb[slot, e, pl.ds(48, 16)]
            msg[e, pl.ds(0, 16)] = _gate(gf0, gc0)
            msg[e, pl.ds(16, 16)] = _gate(gf1, gc1)
        pltpu.sync_copy(msg, acc.at[sidx.at[slot]], add=True)

    issue_idx(0, 0)
    issue_idx(1, 1)
    transform_and_gather(0)

    def pair(p, _):
        for k in (0, 1):
            cg = 2 * p + k
            issue_idx(k, cg + 2)
            transform_and_gather(1 - k)
            compute_and_scatter(k)
        return 0

    lax.fori_loop(0, (nchunks - 1) // 2, pair, 0)
    compute_and_scatter(0)
    wait_idx(1)

    plsc.subcore_barrier()
    pltpu.sync_copy(acc.at[pl.ds(s * ROWS_T, ROWS_T)],
                    out.at[pl.ds(c * NPAD + s * ROWS_T, ROWS_T)])


@functools.cache
def _make_sc_conv(epad):
    nchunks = epad // NSUB // CHUNK
    return pl.kernel(
        functools.partial(_sc_conv_body, nchunks),
        out_type=jax.ShapeDtypeStruct((2 * NPAD, HH), jnp.float32),
        mesh=_mesh(),
        compiler_params=pltpu.CompilerParams(use_tc_tiling_on_sc=False),
        scratch_types=[
            pltpu.VMEM((2, CHUNK), jnp.int32),
            pltpu.VMEM((2, CHUNK), jnp.int32),
            pltpu.VMEM((2, CHUNK), jnp.int32),
            pltpu.VMEM((2, CHUNK), jnp.int32),
            pltpu.VMEM((2, CHUNK), jnp.int32),
            pltpu.VMEM((2, CHUNK, H), jnp.float32),
            pltpu.VMEM((2, CHUNK, H), jnp.float32),
            pltpu.VMEM((CHUNK, HH), jnp.float32),
            pltpu.VMEM_SHARED((NPAD, HH), jnp.float32),
            pltpu.SemaphoreType.DMA,
            pltpu.SemaphoreType.DMA,
            pltpu.SemaphoreType.DMA,
            pltpu.SemaphoreType.DMA,
            pltpu.SemaphoreType.DMA,
            pltpu.SemaphoreType.DMA,
        ],
    )


def _cnt_phase(idx2d, nrows_t, tbase, ones_v, idxbuf, sp):
    nbig, rem = nrows_t // 8, nrows_t % 8

    def big(g, _):
        pltpu.sync_copy(idx2d.at[pl.ds(tbase + g * 8, 8)], idxbuf)
        for j in range(8):
            pltpu.sync_copy(ones_v, sp.at[idxbuf.at[j]], add=True)
        return 0

    lax.fori_loop(0, nbig, big, 0)
    if rem:
        pltpu.sync_copy(idx2d.at[pl.ds(tbase + nbig * 8, rem)],
                        idxbuf.at[pl.ds(0, rem)])
        for j in range(rem):
            pltpu.sync_copy(ones_v, sp.at[idxbuf.at[j]], add=True)


def _sc_cnt_body(nb, nm, bt, ones_hbm, zeros8, cb, cm, cg,
                 idx8, idx8b, ones_v, spn, spg):
    c = lax.axis_index("c")
    s = lax.axis_index("s")
    pltpu.sync_copy(ones_hbm, ones_v)

    @pl.when(c == 0)
    def _():
        pltpu.sync_copy(zeros8, spn.at[pl.ds(s * ROWS_T, ROWS_T)])
        plsc.subcore_barrier()
        nrows = EBPAD // NSUB // CHUNK
        _cnt_phase(nb, nrows, s * nrows, ones_v, idx8, spn)
        plsc.subcore_barrier()
        pltpu.sync_copy(spn.at[pl.ds(s * ROWS_T, ROWS_T)],
                        cb.at[pl.ds(s * ROWS_T, ROWS_T)])

    @pl.when(c == 1)
    def _():
        pltpu.sync_copy(zeros8, spn.at[pl.ds(s * ROWS_T, ROWS_T)])
        plsc.subcore_barrier()
        nrows = EMPAD // NSUB // CHUNK
        _cnt_phase(nm, nrows, s * nrows, ones_v, idx8, spn)
        plsc.subcore_barrier()
        pltpu.sync_copy(spn.at[pl.ds(s * ROWS_T, ROWS_T)],
                        cm.at[pl.ds(s * ROWS_T, ROWS_T)])

        pltpu.sync_copy(zeros8.at[pl.ds(0, GROWS_T)],
                        spg.at[pl.ds(s * GROWS_T, GROWS_T)])
        plsc.subcore_barrier()
        nrows = ROWS_T // 64
        _cnt_phase(bt, nrows, s * nrows, ones_v.at[pl.ds(0, 64)], idx8b, spg)
        plsc.subcore_barrier()
        pltpu.sync_copy(spg.at[pl.ds(s * GROWS_T, GROWS_T)],
                        cg.at[pl.ds(s * GROWS_T, GROWS_T)])


@functools.cache
def _get_sc_cnt():
    return pl.kernel(
        _sc_cnt_body,
        out_type=(jax.ShapeDtypeStruct((NPAD, 8), jnp.float32),
                  jax.ShapeDtypeStruct((NPAD, 8), jnp.float32),
                  jax.ShapeDtypeStruct((GPAD, 8), jnp.float32)),
        mesh=_mesh(),
        compiler_params=pltpu.CompilerParams(use_tc_tiling_on_sc=False),
        scratch_types=[
            pltpu.VMEM((8, CHUNK), jnp.int32),
            pltpu.VMEM((8, 64), jnp.int32),
            pltpu.VMEM((CHUNK, 8), jnp.float32),
            pltpu.VMEM_SHARED((NPAD, 8), jnp.float32),
            pltpu.VMEM_SHARED((GPAD, 8), jnp.float32),
        ],
    )


def _sc_pool_body(xt, bt, zeros32, out, idx64, vbuf, spp):
    c = lax.axis_index("c")
    s = lax.axis_index("s")
    pltpu.sync_copy(zeros32.at[pl.ds(0, GROWS_T)],
                    spp.at[pl.ds(s * GROWS_T, GROWS_T)])
    plsc.subcore_barrier()
    base = s * ROWS_T

    def chunk(g, _):
        off = base + g * 64
        pltpu.sync_copy(bt.at[pl.ds(off, 64)], idx64)
        pltpu.sync_copy(xt.at[pl.ds(c * NPAD + off, 64)], vbuf)
        pltpu.sync_copy(vbuf, spp.at[idx64], add=True)
        return 0

    lax.fori_loop(0, ROWS_T // 64, chunk, 0)
    plsc.subcore_barrier()
    pltpu.sync_copy(spp.at[pl.ds(s * GROWS_T, GROWS_T)],
                    out.at[pl.ds(c * GPAD + s * GROWS_T, GROWS_T)])


@functools.cache
def _get_sc_pool():
    return pl.kernel(
        _sc_pool_body,
        out_type=jax.ShapeDtypeStruct((2 * GPAD, HH), jnp.float32),
        mesh=_mesh(),
        compiler_params=pltpu.CompilerParams(use_tc_tiling_on_sc=False),
        scratch_types=[
            pltpu.VMEM((64,), jnp.int32),
            pltpu.VMEM((64, HH), jnp.float32),
            pltpu.VMEM_SHARED((GPAD, HH), jnp.float32),
        ],
    )


def _embed_atom_body(a_ref, w_ref, b_ref, wt_ref, bt_ref, x_ref, t_ref):
    x = jnp.dot(a_ref[...], w_ref[...],
                preferred_element_type=jnp.float32) + b_ref[...]
    x_ref[...] = x
    p = jnp.dot(x, wt_ref[...], preferred_element_type=jnp.float32) + bt_ref[...]
    t_ref[...] = p.reshape(p.shape[0], 2, H)


def _embed_hedge_body(a_ref, w_ref, b_ref, wb_ref, t0_ref, t1_ref, t2_ref):
    h = jnp.dot(a_ref[...], w_ref[...],
                preferred_element_type=jnp.float32) + b_ref[...]
    for j, t_ref in enumerate((t0_ref, t1_ref, t2_ref)):
        p = jnp.dot(h, wb_ref[j], preferred_element_type=jnp.float32)
        t_ref[...] = p.reshape(p.shape[0], 2, H)


def _update_body(x_ref, a0_ref, a1_ref, cnt_ref, wt_ref, bt_ref,
                 x_out, t_ref):
    aggr = jnp.concatenate([a0_ref[...], a1_ref[...]], axis=1)
    cnt = jnp.maximum(cnt_ref[:, 0:1], 1.0)
    xn = jax.nn.softplus(x_ref[...] + aggr / cnt)
    x_out[...] = xn
    p = jnp.dot(xn, wt_ref[...], preferred_element_type=jnp.float32) + bt_ref[...]
    t_ref[...] = p.reshape(p.shape[0], 2, H)


def _update_last_body(x_ref, a0_ref, a1_ref, cnt_ref, t_ref):
    aggr = jnp.concatenate([a0_ref[...], a1_ref[...]], axis=1)
    cnt = jnp.maximum(cnt_ref[:, 0:1], 1.0)
    xn = jax.nn.softplus(x_ref[...] + aggr / cnt)
    t_ref[0, :, :] = xn[:, 0:HH]
    t_ref[1, :, :] = xn[:, HH:H]


def _head_body(p0_ref, p1_ref, cg_ref, w1, b1, w2, b2, w3, b3, wo, bo, o_ref):
    s = jnp.concatenate([p0_ref[...], p1_ref[...]], axis=1)
    cnt = jnp.maximum(cg_ref[:, 0:1], 1.0)
    h = s / cnt
    h = jax.nn.softplus(jnp.dot(h, w1[...], preferred_element_type=jnp.float32) + b1[...])
    h = jax.nn.softplus(jnp.dot(h, w2[...], preferred_element_type=jnp.float32) + b2[...])
    h = jax.nn.softplus(jnp.dot(h, w3[...], preferred_element_type=jnp.float32) + b3[...])
    o_ref[...] = jnp.dot(h, wo[...], preferred_element_type=jnp.float32) + bo[...]


def _full(shape):
    return pl.BlockSpec(shape, lambda i: tuple(0 for _ in shape))


_BLK = 512


def _embed_atom(attrs, w, b, wt, bt):
    n = attrs.shape[0]
    return pl.pallas_call(
        _embed_atom_body,
        grid=(n // _BLK,),
        in_specs=[
            pl.BlockSpec((_BLK, attrs.shape[1]), lambda i: (i, 0)),
            _full(w.shape), _full(b.shape), _full(wt.shape), _full(bt.shape),
        ],
        out_specs=[
            pl.BlockSpec((_BLK, H), lambda i: (i, 0)),
            pl.BlockSpec((_BLK, 2, H), lambda i: (i, 0, 0)),
        ],
        out_shape=[
            jax.ShapeDtypeStruct((n, H), jnp.float32),
            jax.ShapeDtypeStruct((n, 2, H), jnp.float32),
        ],
    )(attrs, w, b, wt, bt)


def _embed_hedge(attrs, w, b, wb):
    n = attrs.shape[0]
    ts = jax.ShapeDtypeStruct((n, 2, H), jnp.float32)
    return pl.pallas_call(
        _embed_hedge_body,
        grid=(n // _BLK,),
        in_specs=[
            pl.BlockSpec((_BLK, attrs.shape[1]), lambda i: (i, 0)),
            _full(w.shape), _full(b.shape), _full(wb.shape),
        ],
        out_specs=[pl.BlockSpec((_BLK, 2, H), lambda i: (i, 0, 0))] * 3,
        out_shape=[ts, ts, ts],
    )(attrs, w, b, wb)


def _update(x, aggr, cnt, wt, bt):
    nb = NPAD // _BLK
    return pl.pallas_call(
        _update_body,
        grid=(nb,),
        in_specs=[
            pl.BlockSpec((_BLK, H), lambda i: (i, 0)),
            pl.BlockSpec((_BLK, HH), lambda i: (i, 0)),
            pl.BlockSpec((_BLK, HH), lambda i, _n=nb: (i + _n, 0)),
            pl.BlockSpec((_BLK, 8), lambda i: (i, 0)),
            _full(wt.shape), _full(bt.shape),
        ],
        out_specs=[
            pl.BlockSpec((_BLK, H), lambda i: (i, 0)),
            pl.BlockSpec((_BLK, 2, H), lambda i: (i, 0, 0)),
        ],
        out_shape=[
            jax.ShapeDtypeStruct((NPAD, H), jnp.float32),
            jax.ShapeDtypeStruct((NPAD, 2, H), jnp.float32),
        ],
    )(x, aggr, aggr, cnt, wt, bt)


def _update_last(x, aggr, cnt):
    nb = NPAD // _BLK
    return pl.pallas_call(
        _update_last_body,
        grid=(nb,),
        in_specs=[
            pl.BlockSpec((_BLK, H), lambda i: (i, 0)),
            pl.BlockSpec((_BLK, HH), lambda i: (i, 0)),
            pl.BlockSpec((_BLK, HH), lambda i, _n=nb: (i + _n, 0)),
            pl.BlockSpec((_BLK, 8), lambda i: (i, 0)),
        ],
        out_specs=pl.BlockSpec((2, _BLK, HH), lambda i: (0, i, 0)),
        out_shape=jax.ShapeDtypeStruct((2, NPAD, HH), jnp.float32),
    )(x, aggr, aggr, cnt)


def _head(p0, p1, cg, w1, b1, w2, b2, w3, b3, wo, bo):
    return pl.pallas_call(
        _head_body,
        grid=(1,),
        in_specs=[
            pl.BlockSpec((NGRAPH, HH), lambda i: (0, 0)),
            pl.BlockSpec((NGRAPH, HH), lambda i: (0, 0)),
            pl.BlockSpec((NGRAPH, 8), lambda i: (0, 0)),
            _full(w1.shape), _full(b1.shape), _full(w2.shape), _full(b2.shape),
            _full(w3.shape), _full(b3.shape), _full(wo.shape), _full(bo.shape),
        ],
        out_specs=pl.BlockSpec((NGRAPH, 1), lambda i: (0, 0)),
        out_shape=jax.ShapeDtypeStruct((NGRAPH, 1), jnp.float32),
    )(p0, p1, cg, w1, b1, w2, b2, w3, b3, wo, bo)


def _pad_rows(a, n, val=0):
    return jnp.pad(a, ((0, n - a.shape[0]),) + ((0, 0),) * (a.ndim - 1),
                   constant_values=val)


def kernel(atom_attrs, bond_attrs, motif_attrs, bond_index, motif_index,
           batch, W_embed, b_embed, W_bembed, b_bembed, W_membed, b_membed,
           conv_Wf, conv_bf, conv_Wc, conv_bc,
           W1, b1, W2, b2, W3, b3, Wout, bout):
    f32 = jnp.float32
    aa = jnp.pad(atom_attrs, ((0, NPAD - NATOM), (0, 4)))
    ma = jnp.pad(motif_attrs, ((0, NMPAD - NMOTIF), (0, 2)))
    ba = _pad_rows(bond_attrs, NBPAD)
    we = jnp.pad(W_embed, ((0, 4), (0, 0)))
    wm = jnp.pad(W_membed, ((0, 2), (0, 0)))

    hi_b = _pad_rows(bond_index[0], EBPAD + IDXPAD, NBOND)
    ni_b = _pad_rows(bond_index[1], EBPAD + IDXPAD, NATOM)
    hi_m = _pad_rows(motif_index[0], EMPAD + IDXPAD, NMOTIF)
    ni_m = _pad_rows(motif_index[1], EMPAD + IDXPAD, NATOM)
    bt = _pad_rows(batch, NPAD, NGRAPH)

    perm = jnp.concatenate([jnp.arange(HH), jnp.arange(H, H + HH),
                            jnp.arange(HH, H), jnp.arange(H + HH, 2 * H)])
    wtop = [jnp.concatenate([conv_Wf[i][:H], conv_Wc[i][:H]], axis=1)[:, perm]
            for i in range(6)]
    btop = [jnp.concatenate([conv_bf[i], conv_bc[i]])[perm] for i in range(6)]
    wbot = [jnp.concatenate([conv_Wf[i][H:], conv_Wc[i][H:]], axis=1)[:, perm]
            for i in range(6)]

    zeros32 = jnp.zeros((ROWS_T, HH), f32)
    zeros8 = jnp.zeros((ROWS_T, 8), f32)
    ones8 = jnp.ones((CHUNK, 8), f32)

    cb, cm, cg = _get_sc_cnt()(ni_b[:EBPAD].reshape(-1, CHUNK),
                               ni_m[:EMPAD].reshape(-1, CHUNK),
                               bt.reshape(-1, 64), ones8, zeros8)
    x, t = _embed_atom(aa, we, b_embed.reshape(1, -1),
                       wtop[0], btop[0].reshape(1, -1))

    conv_b = _make_sc_conv(EBPAD)
    conv_m = _make_sc_conv(EMPAD)

    tb = _embed_hedge(ba, W_bembed, b_bembed.reshape(1, -1),
                      jnp.stack([wbot[i] for i in (0, 2, 4)]))
    tm = _embed_hedge(ma, wm, b_membed.reshape(1, -1),
                      jnp.stack([wbot[i] for i in (1, 3, 5)]))

    for l in range(6):
        if l % 2 == 0:
            ht = tb[l // 2].reshape(2 * NBPAD, H)
            aggr = conv_b(t.reshape(2 * NPAD, H), ht, ni_b, hi_b, zeros32)
            cnt = cb
        else:
            ht = tm[l // 2].reshape(2 * NMPAD, H)
            aggr = conv_m(t.reshape(2 * NPAD, H), ht, ni_m, hi_m, zeros32)
            cnt = cm
        if l < 5:
            x, t = _update(x, aggr, cnt, wtop[l + 1],
                           btop[l + 1].reshape(1, -1))
        else:
            xt = _update_last(x, aggr, cnt)

    pooled = _get_sc_pool()(xt.reshape(2 * NPAD, HH), bt, zeros32)
    return _head(pooled[:NGRAPH], pooled[GPAD:GPAD + NGRAPH], cg[:NGRAPH],
                 W1, b1.reshape(1, -1), W2, b2.reshape(1, -1),
                 W3, b3.reshape(1, -1), Wout, bout.reshape(1, -1))
